# baseline (speedup 1.0000x reference)
import functools

import jax
import jax.numpy as jnp
from jax import lax
from jax.experimental import pallas as pl
from jax.experimental.pallas import tpu as pltpu
from jax.experimental.pallas import tpu_sc as plsc

_B = 4096
_L = 26
_EDIM = 16
_H1 = 10
_H2 = 5
_NC = 2
_NS = 16
_NW = _NC * _NS
_BPW = _B // _NW
_RPW = _BPW * _L
_GRP = _BPW // _EDIM


def _sc_body(idx_hbm, table_hbm, v_hbm, c_hbm, out_hbm,
             idx_v, rows_v, out_v, v_ref, c_ref, tmp_v, sem):
    wid = lax.axis_index("s") * _NC + lax.axis_index("c")

    pltpu.sync_copy(idx_hbm.at[wid], idx_v)
    pltpu.sync_copy(v_hbm, v_ref)
    pltpu.sync_copy(c_hbm, c_ref)

    copies = []
    for j in range(_L):
        copies.append(pltpu.async_copy(
            table_hbm.at[idx_v.at[j]],
            rows_v.at[pl.ds(j * _BPW, _BPW)],
            sem))
    for c in copies:
        c.wait()

    vvecs = [v_ref[l] for l in range(_L)]
    lanes = lax.iota(jnp.int32, _EDIM)
    cvec = c_ref[...]
    def body(g, carry):
        out16 = cvec
        for k in range(_EDIM):
            base = (g * _EDIM + k) * _L
            acc = rows_v[base] * vvecs[0]
            for l in range(1, _L):
                acc = acc + rows_v[base + l] * vvecs[l]
            total = acc[0]
            for d in range(1, _EDIM):
                total = total + acc[d]
            out16 = jnp.where(lanes == k, out16 + total, out16)
        z = 1.0 / (1.0 + jnp.exp(-out16))
        out_v[pl.ds(g * _EDIM, _EDIM)] = z
        return carry
    lax.fori_loop(0, _GRP, body, 0)

    pltpu.sync_copy(out_v, out_hbm.at[pl.ds(wid * _BPW, _BPW)])


@jax.jit
def _run(idx, table, v, cvec):
    call = functools.partial(
        pl.kernel,
        out_type=jax.ShapeDtypeStruct((_B,), jnp.float32),
        mesh=plsc.VectorSubcoreMesh(core_axis_name="c", subcore_axis_name="s"),
        compiler_params=pltpu.CompilerParams(use_tc_tiling_on_sc=False),
        scratch_types=[
            pltpu.VMEM((_L, _BPW), jnp.int32),
            pltpu.VMEM((_RPW, _EDIM), jnp.float32),
            pltpu.VMEM((_BPW,), jnp.float32),
            pltpu.VMEM((_L, _EDIM), jnp.float32),
            pltpu.VMEM((_EDIM,), jnp.float32),
            pltpu.VMEM((_EDIM,), jnp.float32),
            pltpu.SemaphoreType.DMA,
        ],
    )(_sc_body)
    return call(idx, table, v, cvec)


def kernel(inputs, embed_table, W1, b1, W2, b2, W3, b3):
    W12 = jnp.dot(W1, W2)
    W3r = W3.reshape(_L, _H2)
    v = jnp.dot(W3r, W12.T)
    c = jnp.sum(W3r * (jnp.dot(b1, W2) + b2)[None, :]) + b3[0]
    cvec = jnp.full((_EDIM,), c, dtype=jnp.float32)
    idx = inputs.astype(jnp.int32).reshape(_NW, _L, _BPW)
    out = _run(idx, embed_table, v.astype(jnp.float32), cvec)
    return out.reshape(_B, 1)
